# baseline (device time: 15762 ns/iter reference)
import jax
import jax.numpy as jnp
from jax import lax
from jax.experimental import pallas as pl
from jax.experimental.pallas import tpu as pltpu

N_DEV = 8
N_HALF = 2
OFFSETS = (6, 2, 5, 7, 1, 3, 4)


def kernel(x):
    _, m, n = x.shape
    ch = m // N_DEV
    hch = ch // N_HALF

    def body(x_ref, out_ref, xb_ref, comm_ref, send1, recv1, send2, recv2):
        me = lax.axis_index("i")

        xb_ref[...] = x_ref[0].astype(jnp.bfloat16)

        barrier_sem = pltpu.get_barrier_semaphore()
        for d in OFFSETS:
            pl.semaphore_signal(
                barrier_sem, inc=1,
                device_id=(me ^ d,), device_id_type=pl.DeviceIdType.MESH,
            )
        pl.semaphore_wait(barrier_sem, N_DEV - 1)

        sends = []
        for h in range(N_HALF):
            for d in OFFSETS:
                q = me ^ d
                rdma = pltpu.make_async_remote_copy(
                    src_ref=xb_ref.at[pl.ds(q * ch + h * hch, hch), :],
                    dst_ref=comm_ref.at[h, me],
                    send_sem=send1.at[h, d],
                    recv_sem=recv1.at[h, me],
                    device_id=(q,),
                    device_id_type=pl.DeviceIdType.MESH,
                )
                rdma.start()
                sends.append(rdma)

        for h in range(N_HALF):
            comm_ref[h, pl.ds(me, 1)] = (
                xb_ref[pl.ds(me * ch + h * hch, hch), :][None]
            )

        for h in range(N_HALF):
            for d in OFFSETS:
                p = me ^ d
                recv = pltpu.make_async_remote_copy(
                    src_ref=comm_ref.at[h, p],
                    dst_ref=comm_ref.at[h, p],
                    send_sem=send1.at[h, d],
                    recv_sem=recv1.at[h, p],
                    device_id=(p,),
                    device_id_type=pl.DeviceIdType.MESH,
                )
                recv.wait_recv()

            red = comm_ref[h, 0]
            for p in range(1, N_DEV):
                red = red + comm_ref[h, p]
            out_ref[pl.ds(me * ch + h * hch, hch), :] = red

            for d in OFFSETS:
                q = me ^ d
                rdma = pltpu.make_async_remote_copy(
                    src_ref=out_ref.at[pl.ds(me * ch + h * hch, hch), :],
                    dst_ref=out_ref.at[pl.ds(me * ch + h * hch, hch), :],
                    send_sem=send2.at[h, d],
                    recv_sem=recv2.at[h, me],
                    device_id=(q,),
                    device_id_type=pl.DeviceIdType.MESH,
                )
                rdma.start()
                sends.append(rdma)

        for h in range(N_HALF):
            for d in OFFSETS:
                p = me ^ d
                recv = pltpu.make_async_remote_copy(
                    src_ref=out_ref.at[pl.ds(p * ch + h * hch, hch), :],
                    dst_ref=out_ref.at[pl.ds(p * ch + h * hch, hch), :],
                    send_sem=send2.at[h, d],
                    recv_sem=recv2.at[h, p],
                    device_id=(p,),
                    device_id_type=pl.DeviceIdType.MESH,
                )
                recv.wait_recv()

        for s in sends:
            s.wait_send()

    out_shape = jax.ShapeDtypeStruct((m, n), jnp.bfloat16)
    return pl.pallas_call(
        body,
        out_shape=out_shape,
        in_specs=[pl.BlockSpec(memory_space=pltpu.VMEM)],
        out_specs=pl.BlockSpec(memory_space=pltpu.VMEM),
        scratch_shapes=[
            pltpu.VMEM((m, n), jnp.bfloat16),
            pltpu.VMEM((N_HALF, N_DEV, hch, n), jnp.bfloat16),
            pltpu.SemaphoreType.DMA((N_HALF, N_DEV)),
            pltpu.SemaphoreType.DMA((N_HALF, N_DEV)),
            pltpu.SemaphoreType.DMA((N_HALF, N_DEV)),
            pltpu.SemaphoreType.DMA((N_HALF, N_DEV)),
        ],
        compiler_params=pltpu.CompilerParams(collective_id=0),
    )(x)
